# baseline (speedup 1.0000x reference)
import functools

import jax
import jax.numpy as jnp
from jax import lax
from jax.experimental import pallas as pl
from jax.experimental.pallas import tpu as pltpu
from jax.experimental.pallas import tpu_sc as plsc

H = 128
LEVEL_NS = [10000, 2500, 625, 156, 4]
NCORES = 2
NSUB = 16
NWORKERS = NCORES * NSUB
CHUNK = 128


def _round_up(a, b):
    return -(-a // b) * b


NBUF = 6
PF_DEPTH = 3


@functools.lru_cache(maxsize=None)
def _make_segsum(n_in, n_out_pad, e_pad, chunk):
    m = e_pad // NWORKERS
    n_chunks = m // chunk
    stripe = n_out_pad // NSUB
    mesh = plsc.VectorSubcoreMesh(core_axis_name="c", subcore_axis_name="s")

    @functools.partial(
        pl.kernel,
        mesh=mesh,
        out_type=jax.ShapeDtypeStruct((2 * n_out_pad, H), jnp.float32),
        scratch_types=[
            pltpu.VMEM((n_chunks, 2, chunk), jnp.int32),
            pltpu.VMEM((NBUF, chunk, H), jnp.float32),
            pltpu.VMEM_SHARED((n_out_pad, H), jnp.float32),
        ] + [pltpu.SemaphoreType.DMA] * (2 * NBUF),
    )
    def seg_kernel(z_hbm, sd_hbm, zero_hbm, out_hbm,
                   idx_v, rows_v, acc_sh, *sems):
        gsem, ssem = sems[:NBUF], sems[NBUF:]
        c = lax.axis_index("c")
        s = lax.axis_index("s")
        wid = s * NCORES + c
        r0 = s * stripe
        pltpu.sync_copy(sd_hbm.at[pl.ds(wid * n_chunks, n_chunks)], idx_v)
        pf_depth = min(PF_DEPTH, n_chunks)
        gathers = [None] * NBUF
        scatters = [None] * NBUF
        for b in range(pf_depth):
            gathers[b] = pltpu.async_copy(
                z_hbm.at[idx_v.at[b, 0]], rows_v.at[b], gsem[b])
        pltpu.sync_copy(zero_hbm.at[pl.ds(r0, stripe)],
                        acc_sh.at[pl.ds(r0, stripe)])
        plsc.subcore_barrier()
        for j in range(n_chunks):
            b = j % NBUF
            gathers[b].wait()
            scatters[b] = pltpu.async_copy(
                rows_v.at[b], acc_sh.at[idx_v.at[j, 1]], ssem[b], add=True)
            pf = j + pf_depth
            if pf < n_chunks:
                pb = pf % NBUF
                if pf >= NBUF:
                    scatters[pb].wait()
                gathers[pb] = pltpu.async_copy(
                    z_hbm.at[idx_v.at[pf, 0]], rows_v.at[pb], gsem[pb])
        for k in range(max(0, n_chunks - NBUF), n_chunks):
            scatters[k % NBUF].wait()
        plsc.subcore_barrier()
        pltpu.sync_copy(acc_sh.at[pl.ds(r0, stripe)],
                        out_hbm.at[pl.ds(c * n_out_pad + r0, stripe)])

    return seg_kernel


def _segsum(z, sd, zeros, n_out_pad):
    chunk = sd.shape[2]
    e_pad = sd.shape[0] * chunk
    return _make_segsum(z.shape[0], n_out_pad, e_pad, chunk)(z, sd, zeros)


def _chunk_for(e):
    del e
    return CHUNK


def _pad_idx(src, dst, n_in, n_out, n_out_pad):
    e = src.shape[0]
    chunk = _chunk_for(e)
    e_pad = _round_up(max(e, 1), NWORKERS * chunk)
    pad = e_pad - e
    src = src.astype(jnp.int32)
    dst = dst.astype(jnp.int32)
    if pad:
        r = jnp.arange(pad, dtype=jnp.int32)
        src = jnp.concatenate([src, r % n_in])
        dst = jnp.concatenate([dst, n_out + r % (n_out_pad - n_out)])
    return jnp.stack([src.reshape(-1, chunk), dst.reshape(-1, chunk)], axis=1)


def _dot(a, b):
    return jnp.dot(a, b, preferred_element_type=jnp.float32)


def _first_body(x_ref, w_ref, b_ref, hx_ref, cxs_ref):
    x = x_ref[...]
    hx = _dot(x, w_ref[...]) + b_ref[...]
    hx_ref[...] = hx
    g = 2.0 * jax.nn.sigmoid(jnp.sum(hx, axis=1, keepdims=True))
    cxs_ref[...] = g * x


def _a_body(p_ref, wd_ref, bd_ref, wr1_ref, u0_ref, z1_ref):
    n = u0_ref.shape[0]
    ph = p_ref[:n, :] + p_ref[n:, :]
    u0 = jnp.maximum(_dot(ph, wd_ref[...]) + bd_ref[...], 0.0)
    u0_ref[...] = u0
    z1_ref[...] = _dot(u0, wr1_ref[...])


def _b_body(s_ref, b_ref, w_ref, z_ref):
    n = z_ref.shape[0]
    y = jnp.maximum(s_ref[:n, :] + s_ref[n:, :] + b_ref[...], 0.0)
    z_ref[...] = _dot(y, w_ref[...])


def _c_body(s_ref, b_ref, u0_ref, w_ref, z_ref):
    n = z_ref.shape[0]
    u1 = jnp.maximum(u0_ref[...] + s_ref[:n, :] + s_ref[n:, :] + b_ref[...],
                     0.0)
    z_ref[...] = _dot(u1, w_ref[...])


def _d_gate_body(s_ref, b_ref, v_ref, cxs_ref):
    n = v_ref.shape[0]
    v = jnp.maximum(s_ref[:n, :] + s_ref[n:, :] + b_ref[...], 0.0)
    v_ref[...] = v
    g = 2.0 * jax.nn.sigmoid(jnp.sum(v, axis=1, keepdims=True))
    cxs_ref[...] = g * v


def _d_body(s_ref, b_ref, v_ref):
    n = v_ref.shape[0]
    v_ref[...] = jnp.maximum(s_ref[:n, :] + s_ref[n:, :] + b_ref[...], 0.0)


def _dense_level_body(with_gate, cur_ref, pool_ref, src_ref, dst_ref,
                      wd_ref, bd_ref, wr1_ref, br1_ref, wr2_ref, br2_ref,
                      wc_ref, bc_ref, v_ref, *maybe_cxs):
    np_ = v_ref.shape[0]
    iota_c = lax.broadcasted_iota(jnp.int32, (np_, 1), 0)
    pool = (pool_ref[...] == iota_c).astype(jnp.float32)
    pooled = _dot(pool, cur_ref[...])
    u0 = jnp.maximum(_dot(pooled, wd_ref[...]) + bd_ref[...], 0.0)
    od = (dst_ref[...] == iota_c).astype(jnp.float32)
    os = (src_ref[...] == iota_c).astype(jnp.float32)
    amat = jax.lax.dot_general(od, os, (((1,), (1,)), ((), ())),
                               preferred_element_type=jnp.float32)
    y = jnp.maximum(_dot(amat, _dot(u0, wr1_ref[...])) + br1_ref[...], 0.0)
    y2 = _dot(amat, _dot(y, wr2_ref[...])) + br2_ref[...]
    u1 = jnp.maximum(u0 + y2, 0.0)
    v = jnp.maximum(_dot(amat, _dot(u1, wc_ref[...])) + bc_ref[...], 0.0)
    v_ref[...] = v
    if with_gate:
        g = 2.0 * jax.nn.sigmoid(jnp.sum(v, axis=1, keepdims=True))
        maybe_cxs[0][...] = g * v


def _tc(body, out_shapes, *args):
    multi = isinstance(out_shapes[0], tuple)
    shapes = out_shapes if multi else (out_shapes,)
    out = pl.pallas_call(
        body,
        out_shape=tuple(jax.ShapeDtypeStruct(s, jnp.float32)
                        for s in shapes))(*args)
    return out if multi else out[0]


def kernel(x, params, pool0, edge_index1, pool1, edge_index2, pool2,
           edge_index3, pool3, edge_index4, batch_size):
    del batch_size
    pools = [pool0, pool1, pool2, pool3]
    edges = [edge_index1, edge_index2, edge_index3, edge_index4]

    b2 = {k: v.reshape(1, H) for k, v in params.items() if v.ndim == 1}

    hx, cxs = _tc(_first_body, ((LEVEL_NS[0], H), (LEVEL_NS[0], H)),
                  x, params['W_first'], b2['b_first'])
    outs = [hx]
    cur = cxs
    for i in range(4):
        nf = LEVEL_NS[i]
        nc = LEVEL_NS[i + 1]
        if i >= 2:
            np_ = _round_up(nc, 8)
            nfp = cur.shape[0]
            poolp = jnp.full((1, nfp), np_, jnp.int32)
            poolp = poolp.at[0, :nf].set(pools[i].astype(jnp.int32))
            e = edges[i].shape[1]
            ep = _round_up(e, 8)
            eidx = jnp.full((2, ep), np_, jnp.int32)
            eidx = eidx.at[:, :e].set(edges[i].astype(jnp.int32))
            args = (cur, poolp, eidx[0:1], eidx[1:2],
                    params[f'Wd{i}'], b2[f'bd{i}'],
                    params[f'Wr1{i}'], b2[f'br1{i}'],
                    params[f'Wr2{i}'], b2[f'br2{i}'],
                    params[f'Wc{i}'], b2[f'bc{i}'])
            if i < 3:
                v, cxs = _tc(functools.partial(_dense_level_body, True),
                             ((np_, H), (np_, H)), *args)
                cur = cxs
            else:
                v = _tc(functools.partial(_dense_level_body, False),
                        (np_, H), *args)
            outs.append(v[:nc])
            continue
        ncp = max(512, _round_up(nc + 1, NSUB * 8))
        zeros = jnp.zeros((ncp, H), jnp.float32)
        psd = _pad_idx(jnp.arange(nf, dtype=jnp.int32), pools[i],
                       cur.shape[0], nc, ncp)
        p = _segsum(cur, psd, zeros, ncp)
        u0, z1 = _tc(_a_body, ((ncp, H), (ncp, H)),
                     p, params[f'Wd{i}'], b2[f'bd{i}'], params[f'Wr1{i}'])
        esd = _pad_idx(edges[i][0], edges[i][1], ncp, nc, ncp)
        s1 = _segsum(z1, esd, zeros, ncp)
        z2 = _tc(_b_body, (ncp, H), s1, b2[f'br1{i}'], params[f'Wr2{i}'])
        s2 = _segsum(z2, esd, zeros, ncp)
        z3 = _tc(_c_body, (ncp, H), s2, b2[f'br2{i}'], u0, params[f'Wc{i}'])
        s3 = _segsum(z3, esd, zeros, ncp)
        if i < 3:
            v, cxs = _tc(_d_gate_body, ((ncp, H), (ncp, H)), s3, b2[f'bc{i}'])
            cur = cxs
        else:
            v = _tc(_d_body, (ncp, H), s3, b2[f'bc{i}'])
        outs.append(v[:nc])
    return tuple(outs)


# speedup vs baseline: 1.0119x; 1.0119x over previous
import functools

import jax
import jax.numpy as jnp
from jax import lax
from jax.experimental import pallas as pl
from jax.experimental.pallas import tpu as pltpu
from jax.experimental.pallas import tpu_sc as plsc

H = 128
LEVEL_NS = [10000, 2500, 625, 156, 4]
NCORES = 2
NSUB = 16
NWORKERS = NCORES * NSUB
CHUNK = 128


def _round_up(a, b):
    return -(-a // b) * b


NBUF = 4
PF_DEPTH = 2


@functools.lru_cache(maxsize=None)
def _make_segsum(n_in, n_out_pad, e_pad, chunk):
    m = e_pad // NWORKERS
    n_chunks = m // chunk
    stripe = n_out_pad // NSUB
    mesh = plsc.VectorSubcoreMesh(core_axis_name="c", subcore_axis_name="s")

    @functools.partial(
        pl.kernel,
        mesh=mesh,
        out_type=jax.ShapeDtypeStruct((2 * n_out_pad, H), jnp.float32),
        scratch_types=[
            pltpu.VMEM((n_chunks, 2, chunk), jnp.int32),
            pltpu.VMEM((NBUF, chunk, H), jnp.float32),
            pltpu.VMEM_SHARED((n_out_pad, H), jnp.float32),
        ] + [pltpu.SemaphoreType.DMA] * (2 * NBUF),
    )
    def seg_kernel(z_hbm, sd_hbm, zero_hbm, out_hbm,
                   idx_v, rows_v, acc_sh, *sems):
        gsem, ssem = sems[:NBUF], sems[NBUF:]
        c = lax.axis_index("c")
        s = lax.axis_index("s")
        wid = s * NCORES + c
        r0 = s * stripe
        pltpu.sync_copy(sd_hbm.at[pl.ds(wid * n_chunks, n_chunks)], idx_v)
        pf_depth = min(PF_DEPTH, n_chunks)
        gathers = [None] * NBUF
        scatters = [None] * NBUF
        for b in range(pf_depth):
            gathers[b] = pltpu.async_copy(
                z_hbm.at[idx_v.at[b, 0]], rows_v.at[b], gsem[b])
        pltpu.sync_copy(zero_hbm.at[pl.ds(r0, stripe)],
                        acc_sh.at[pl.ds(r0, stripe)])
        plsc.subcore_barrier()
        for j in range(n_chunks):
            b = j % NBUF
            gathers[b].wait()
            scatters[b] = pltpu.async_copy(
                rows_v.at[b], acc_sh.at[idx_v.at[j, 1]], ssem[b], add=True)
            pf = j + pf_depth
            if pf < n_chunks:
                pb = pf % NBUF
                if pf >= NBUF:
                    scatters[pb].wait()
                gathers[pb] = pltpu.async_copy(
                    z_hbm.at[idx_v.at[pf, 0]], rows_v.at[pb], gsem[pb])
        for k in range(max(0, n_chunks - NBUF), n_chunks):
            scatters[k % NBUF].wait()
        plsc.subcore_barrier()
        pltpu.sync_copy(acc_sh.at[pl.ds(r0, stripe)],
                        out_hbm.at[pl.ds(c * n_out_pad + r0, stripe)])

    return seg_kernel


def _segsum(z, sd, zeros, n_out_pad):
    chunk = sd.shape[2]
    e_pad = sd.shape[0] * chunk
    return _make_segsum(z.shape[0], n_out_pad, e_pad, chunk)(z, sd, zeros)


def _chunk_for(e):
    del e
    return CHUNK


def _pad_idx(src, dst, n_in, n_out, n_out_pad):
    e = src.shape[0]
    chunk = _chunk_for(e)
    e_pad = _round_up(max(e, 1), NWORKERS * chunk)
    pad = e_pad - e
    src = src.astype(jnp.int32)
    dst = dst.astype(jnp.int32)
    if pad:
        r = jnp.arange(pad, dtype=jnp.int32)
        src = jnp.concatenate([src, r % n_in])
        dst = jnp.concatenate([dst, n_out + r % (n_out_pad - n_out)])
    return jnp.stack([src.reshape(-1, chunk), dst.reshape(-1, chunk)], axis=1)


def _dot(a, b):
    return jnp.dot(a, b, preferred_element_type=jnp.float32)


def _first_body(x_ref, w_ref, b_ref, hx_ref, cxs_ref):
    x = x_ref[...]
    hx = _dot(x, w_ref[...]) + b_ref[...]
    hx_ref[...] = hx
    g = 2.0 * jax.nn.sigmoid(jnp.sum(hx, axis=1, keepdims=True))
    cxs_ref[...] = g * x


def _a_body(p_ref, wd_ref, bd_ref, wr1_ref, u0_ref, z1_ref):
    n = u0_ref.shape[0]
    ph = p_ref[:n, :] + p_ref[n:, :]
    u0 = jnp.maximum(_dot(ph, wd_ref[...]) + bd_ref[...], 0.0)
    u0_ref[...] = u0
    z1_ref[...] = _dot(u0, wr1_ref[...])


def _b_body(s_ref, b_ref, w_ref, z_ref):
    n = z_ref.shape[0]
    y = jnp.maximum(s_ref[:n, :] + s_ref[n:, :] + b_ref[...], 0.0)
    z_ref[...] = _dot(y, w_ref[...])


def _c_body(s_ref, b_ref, u0_ref, w_ref, z_ref):
    n = z_ref.shape[0]
    u1 = jnp.maximum(u0_ref[...] + s_ref[:n, :] + s_ref[n:, :] + b_ref[...],
                     0.0)
    z_ref[...] = _dot(u1, w_ref[...])


def _d_gate_body(s_ref, b_ref, v_ref, cxs_ref):
    n = v_ref.shape[0]
    v = jnp.maximum(s_ref[:n, :] + s_ref[n:, :] + b_ref[...], 0.0)
    v_ref[...] = v
    g = 2.0 * jax.nn.sigmoid(jnp.sum(v, axis=1, keepdims=True))
    cxs_ref[...] = g * v


def _d_body(s_ref, b_ref, v_ref):
    n = v_ref.shape[0]
    v_ref[...] = jnp.maximum(s_ref[:n, :] + s_ref[n:, :] + b_ref[...], 0.0)


def _gate_scale(v):
    return 2.0 * jax.nn.sigmoid(jnp.sum(v, axis=1, keepdims=True)) * v


def _dense_level(cur, pool_row, src_row, dst_row, np_, w):
    wd, bd, wr1, br1, wr2, br2, wc, bc = w
    iota_c = lax.broadcasted_iota(jnp.int32, (np_, 1), 0)
    pool = (pool_row == iota_c).astype(jnp.float32)
    pooled = _dot(pool, cur)
    u0 = jnp.maximum(_dot(pooled, wd) + bd, 0.0)
    od = (dst_row == iota_c).astype(jnp.float32)
    os = (src_row == iota_c).astype(jnp.float32)
    amat = jax.lax.dot_general(od, os, (((1,), (1,)), ((), ())),
                               preferred_element_type=jnp.float32)
    y = jnp.maximum(_dot(amat, _dot(u0, wr1)) + br1, 0.0)
    y2 = _dot(amat, _dot(y, wr2)) + br2
    u1 = jnp.maximum(u0 + y2, 0.0)
    return jnp.maximum(_dot(amat, _dot(u1, wc)) + bc, 0.0)


def _tail_body(s_ref, bc1_ref, pool2_ref, src3_ref, dst3_ref,
               pool3_ref, src4_ref, dst4_ref,
               w2a, w2b, w2c, w2d, w2e, w2f, w2g, w2h,
               w3a, w3b, w3c, w3d, w3e, w3f, w3g, w3h,
               v1_ref, v2_ref, v3_ref):
    n = v1_ref.shape[0]
    v1 = jnp.maximum(s_ref[:n, :] + s_ref[n:, :] + bc1_ref[...], 0.0)
    v1_ref[...] = v1
    w2 = tuple(r[...] for r in (w2a, w2b, w2c, w2d, w2e, w2f, w2g, w2h))
    v2 = _dense_level(_gate_scale(v1), pool2_ref[...], src3_ref[...],
                      dst3_ref[...], v2_ref.shape[0], w2)
    v2_ref[...] = v2
    w3 = tuple(r[...] for r in (w3a, w3b, w3c, w3d, w3e, w3f, w3g, w3h))
    v3_ref[...] = _dense_level(_gate_scale(v2), pool3_ref[...], src4_ref[...],
                               dst4_ref[...], v3_ref.shape[0], w3)


def _tc(body, out_shapes, *args):
    multi = isinstance(out_shapes[0], tuple)
    shapes = out_shapes if multi else (out_shapes,)
    out = pl.pallas_call(
        body,
        out_shape=tuple(jax.ShapeDtypeStruct(s, jnp.float32)
                        for s in shapes))(*args)
    return out if multi else out[0]


def kernel(x, params, pool0, edge_index1, pool1, edge_index2, pool2,
           edge_index3, pool3, edge_index4, batch_size):
    del batch_size
    pools = [pool0, pool1, pool2, pool3]
    edges = [edge_index1, edge_index2, edge_index3, edge_index4]

    b2 = {k: v.reshape(1, H) for k, v in params.items() if v.ndim == 1}

    hx, cxs = _tc(_first_body, ((LEVEL_NS[0], H), (LEVEL_NS[0], H)),
                  x, params['W_first'], b2['b_first'])
    outs = [hx]
    cur = cxs
    s3 = None
    for i in range(2):
        nf = LEVEL_NS[i]
        nc = LEVEL_NS[i + 1]
        ncp = max(512, _round_up(nc + 1, NSUB * 8))
        zeros = jnp.zeros((ncp, H), jnp.float32)
        psd = _pad_idx(jnp.arange(nf, dtype=jnp.int32), pools[i],
                       cur.shape[0], nc, ncp)
        p = _segsum(cur, psd, zeros, ncp)
        u0, z1 = _tc(_a_body, ((ncp, H), (ncp, H)),
                     p, params[f'Wd{i}'], b2[f'bd{i}'], params[f'Wr1{i}'])
        esd = _pad_idx(edges[i][0], edges[i][1], ncp, nc, ncp)
        s1 = _segsum(z1, esd, zeros, ncp)
        z2 = _tc(_b_body, (ncp, H), s1, b2[f'br1{i}'], params[f'Wr2{i}'])
        s2 = _segsum(z2, esd, zeros, ncp)
        z3 = _tc(_c_body, (ncp, H), s2, b2[f'br2{i}'], u0, params[f'Wc{i}'])
        s3 = _segsum(z3, esd, zeros, ncp)
        if i == 0:
            v, cxs = _tc(_d_gate_body, ((ncp, H), (ncp, H)), s3, b2[f'bc{i}'])
            cur = cxs
            outs.append(v[:nc])
            ncp1 = None
        else:
            ncp1 = ncp

    def pad_row(a, n, fill):
        out = jnp.full((1, n), fill, jnp.int32)
        return out.at[0, :a.shape[0]].set(a.astype(jnp.int32))

    np2, np3 = _round_up(LEVEL_NS[3], 8), _round_up(LEVEL_NS[4], 8)
    e3, e4 = edges[2].astype(jnp.int32), edges[3].astype(jnp.int32)
    v1, v2, v3 = _tc(
        _tail_body, ((ncp1, H), (np2, H), (np3, H)),
        s3, b2['bc1'],
        pad_row(pools[2], ncp1, np2), e3[0:1], e3[1:2],
        pad_row(pools[3], np2, np3), e4[0:1], e4[1:2],
        params['Wd2'], b2['bd2'], params['Wr12'], b2['br12'],
        params['Wr22'], b2['br22'], params['Wc2'], b2['bc2'],
        params['Wd3'], b2['bd3'], params['Wr13'], b2['br13'],
        params['Wr23'], b2['br23'], params['Wc3'], b2['bc3'])
    outs.extend([v1[:LEVEL_NS[2]], v2[:LEVEL_NS[3]], v3[:LEVEL_NS[4]]])
    return tuple(outs)


# speedup vs baseline: 1.0124x; 1.0005x over previous
import functools

import jax
import jax.numpy as jnp
from jax import lax
from jax.experimental import pallas as pl
from jax.experimental.pallas import tpu as pltpu
from jax.experimental.pallas import tpu_sc as plsc

H = 128
LEVEL_NS = [10000, 2500, 625, 156, 4]
NCORES = 2
NSUB = 16
NWORKERS = NCORES * NSUB
CHUNK = 128


def _round_up(a, b):
    return -(-a // b) * b


NBUF = 4
PF_DEPTH = 2


@functools.lru_cache(maxsize=None)
def _make_segsum(n_in, n_out_pad, e_pad, chunk):
    m = e_pad // NWORKERS
    n_chunks = m // chunk
    stripe = n_out_pad // NSUB
    mesh = plsc.VectorSubcoreMesh(core_axis_name="c", subcore_axis_name="s")

    @functools.partial(
        pl.kernel,
        mesh=mesh,
        out_type=jax.ShapeDtypeStruct((2 * n_out_pad, H), jnp.float32),
        scratch_types=[
            pltpu.VMEM((n_chunks, 2, chunk), jnp.int32),
            pltpu.VMEM((NBUF, chunk, H), jnp.float32),
            pltpu.VMEM_SHARED((n_out_pad, H), jnp.float32),
        ] + [pltpu.SemaphoreType.DMA] * (2 * NBUF),
    )
    def seg_kernel(z_hbm, sd_hbm, zero_hbm, out_hbm,
                   idx_v, rows_v, acc_sh, *sems):
        gsem, ssem = sems[:NBUF], sems[NBUF:]
        c = lax.axis_index("c")
        s = lax.axis_index("s")
        wid = s * NCORES + c
        r0 = s * stripe
        pltpu.sync_copy(sd_hbm.at[pl.ds(wid * n_chunks, n_chunks)], idx_v)
        pf_depth = min(PF_DEPTH, n_chunks)
        gathers = [None] * NBUF
        scatters = [None] * NBUF
        for b in range(pf_depth):
            gathers[b] = pltpu.async_copy(
                z_hbm.at[idx_v.at[b, 0]], rows_v.at[b], gsem[b])
        pltpu.sync_copy(zero_hbm.at[pl.ds(r0, stripe)],
                        acc_sh.at[pl.ds(r0, stripe)])
        plsc.subcore_barrier()
        for j in range(n_chunks):
            b = j % NBUF
            gathers[b].wait()
            scatters[b] = pltpu.async_copy(
                rows_v.at[b], acc_sh.at[idx_v.at[j, 1]], ssem[b], add=True)
            pf = j + pf_depth
            if pf < n_chunks:
                pb = pf % NBUF
                if pf >= NBUF:
                    scatters[pb].wait()
                gathers[pb] = pltpu.async_copy(
                    z_hbm.at[idx_v.at[pf, 0]], rows_v.at[pb], gsem[pb])
        for k in range(max(0, n_chunks - NBUF), n_chunks):
            scatters[k % NBUF].wait()
        plsc.subcore_barrier()
        pltpu.sync_copy(acc_sh.at[pl.ds(r0, stripe)],
                        out_hbm.at[pl.ds(c * n_out_pad + r0, stripe)])

    return seg_kernel


def _segsum(z, sd, zeros, n_out_pad):
    chunk = sd.shape[2]
    e_pad = sd.shape[0] * chunk
    return _make_segsum(z.shape[0], n_out_pad, e_pad, chunk)(z, sd, zeros)


def _chunk_for(e):
    del e
    return CHUNK


def _pad_idx(src, dst, n_in, n_out, n_out_pad):
    e = src.shape[0]
    chunk = _chunk_for(e)
    e_pad = _round_up(max(e, 1), NWORKERS * chunk)
    pad = e_pad - e
    src = src.astype(jnp.int32)
    dst = dst.astype(jnp.int32)
    if pad:
        r = jnp.arange(pad, dtype=jnp.int32)
        src = jnp.concatenate([src, r % n_in])
        dst = jnp.concatenate([dst, n_out + r % (n_out_pad - n_out)])
    return jnp.stack([src.reshape(-1, chunk), dst.reshape(-1, chunk)], axis=1)


def _dot(a, b):
    return jnp.dot(a, b, preferred_element_type=jnp.float32)


def _first_body(x_ref, w_ref, b_ref, hx_ref, cxs_ref):
    x = x_ref[...]
    hx = _dot(x, w_ref[...]) + b_ref[...]
    hx_ref[...] = hx
    g = 2.0 * jax.nn.sigmoid(jnp.sum(hx, axis=1, keepdims=True))
    cxs_ref[...] = g * x


def _a_body(p_ref, wd_ref, bd_ref, wr1_ref, u0_ref, z1_ref):
    n = u0_ref.shape[0]
    ph = p_ref[:n, :] + p_ref[n:, :]
    u0 = jnp.maximum(_dot(ph, wd_ref[...]) + bd_ref[...], 0.0)
    u0_ref[...] = u0
    z1_ref[...] = _dot(u0, wr1_ref[...])


def _b_body(s_ref, b_ref, w_ref, z_ref):
    n = z_ref.shape[0]
    y = jnp.maximum(s_ref[:n, :] + s_ref[n:, :] + b_ref[...], 0.0)
    z_ref[...] = _dot(y, w_ref[...])


def _c_body(s_ref, b_ref, u0_ref, w_ref, z_ref):
    n = z_ref.shape[0]
    u1 = jnp.maximum(u0_ref[...] + s_ref[:n, :] + s_ref[n:, :] + b_ref[...],
                     0.0)
    z_ref[...] = _dot(u1, w_ref[...])


def _d_gate_body(s_ref, b_ref, v_ref, cxs_ref):
    n = v_ref.shape[0]
    v = jnp.maximum(s_ref[:n, :] + s_ref[n:, :] + b_ref[...], 0.0)
    v_ref[...] = v
    g = 2.0 * jax.nn.sigmoid(jnp.sum(v, axis=1, keepdims=True))
    cxs_ref[...] = g * v


def _gate_scale(v):
    return 2.0 * jax.nn.sigmoid(jnp.sum(v, axis=1, keepdims=True)) * v


def _dense_level(cur, pool_row, src_row, dst_row, np_, w):
    wd, bd, wr1, br1, wr2, br2, wc, bc = w
    iota_c = lax.broadcasted_iota(jnp.int32, (np_, 1), 0)
    pool = (pool_row == iota_c).astype(jnp.float32)
    pooled = _dot(pool, cur)
    u0 = jnp.maximum(_dot(pooled, wd) + bd, 0.0)
    od = (dst_row == iota_c).astype(jnp.float32)
    os = (src_row == iota_c).astype(jnp.float32)
    amat = jax.lax.dot_general(od, os, (((1,), (1,)), ((), ())),
                               preferred_element_type=jnp.float32)
    y = jnp.maximum(_dot(amat, _dot(u0, wr1)) + br1, 0.0)
    y2 = _dot(amat, _dot(y, wr2)) + br2
    u1 = jnp.maximum(u0 + y2, 0.0)
    return jnp.maximum(_dot(amat, _dot(u1, wc)) + bc, 0.0)


def _tail_body(s_ref, bc1_ref, pool2_ref, src3_ref, dst3_ref,
               pool3_ref, src4_ref, dst4_ref,
               w2a, w2b, w2c, w2d, w2e, w2f, w2g, w2h,
               w3a, w3b, w3c, w3d, w3e, w3f, w3g, w3h,
               v1_ref, v2_ref, v3_ref):
    n = v1_ref.shape[0]
    v1 = jnp.maximum(s_ref[:n, :] + s_ref[n:, :] + bc1_ref[...], 0.0)
    v1_ref[...] = v1
    w2 = tuple(r[...] for r in (w2a, w2b, w2c, w2d, w2e, w2f, w2g, w2h))
    v2 = _dense_level(_gate_scale(v1), pool2_ref[...], src3_ref[...],
                      dst3_ref[...], v2_ref.shape[0], w2)
    v2_ref[...] = v2
    w3 = tuple(r[...] for r in (w3a, w3b, w3c, w3d, w3e, w3f, w3g, w3h))
    v3_ref[...] = _dense_level(_gate_scale(v2), pool3_ref[...], src4_ref[...],
                               dst4_ref[...], v3_ref.shape[0], w3)


def _tc(body, out_shapes, *args):
    multi = isinstance(out_shapes[0], tuple)
    shapes = out_shapes if multi else (out_shapes,)
    out = pl.pallas_call(
        body,
        out_shape=tuple(jax.ShapeDtypeStruct(s, jnp.float32)
                        for s in shapes))(*args)
    return out if multi else out[0]


def kernel(x, params, pool0, edge_index1, pool1, edge_index2, pool2,
           edge_index3, pool3, edge_index4, batch_size):
    del batch_size
    pools = [pool0, pool1, pool2, pool3]
    edges = [edge_index1, edge_index2, edge_index3, edge_index4]

    b2 = {k: v.reshape(1, H) for k, v in params.items() if v.ndim == 1}

    hx, cxs = _tc(_first_body, ((LEVEL_NS[0], H), (LEVEL_NS[0], H)),
                  x, params['W_first'], b2['b_first'])
    outs = [hx]
    cur = cxs
    s3 = None
    for i in range(2):
        nf = LEVEL_NS[i]
        nc = LEVEL_NS[i + 1]
        ncp = max(512, _round_up(nc + 1, NSUB * 8))
        zeros = jnp.zeros((ncp, H), jnp.float32)
        psd = _pad_idx(jnp.arange(nf, dtype=jnp.int32), pools[i],
                       cur.shape[0], nc, ncp)
        p = _segsum(cur, psd, zeros, ncp)
        u0, z1 = _tc(_a_body, ((ncp, H), (ncp, H)),
                     p, params[f'Wd{i}'], b2[f'bd{i}'], params[f'Wr1{i}'])
        esd = _pad_idx(edges[i][0], edges[i][1], ncp, nc, ncp)
        s1 = _segsum(z1, esd, zeros, ncp)
        z2 = _tc(_b_body, (ncp, H), s1, b2[f'br1{i}'], params[f'Wr2{i}'])
        s2 = _segsum(z2, esd, zeros, ncp)
        z3 = _tc(_c_body, (ncp, H), s2, b2[f'br2{i}'], u0, params[f'Wc{i}'])
        s3 = _segsum(z3, esd, zeros, ncp)
        if i == 0:
            v, cxs = _tc(_d_gate_body, ((ncp, H), (ncp, H)), s3, b2[f'bc{i}'])
            cur = cxs
            outs.append(v[:nc])
            ncp1 = None
        else:
            ncp1 = ncp

    def pad_row(a, n, fill):
        out = jnp.full((1, n), fill, jnp.int32)
        return out.at[0, :a.shape[0]].set(a.astype(jnp.int32))

    np2, np3 = _round_up(LEVEL_NS[3], 8), _round_up(LEVEL_NS[4], 8)
    e3, e4 = edges[2].astype(jnp.int32), edges[3].astype(jnp.int32)
    v1, v2, v3 = _tc(
        _tail_body, ((ncp1, H), (np2, H), (np3, H)),
        s3, b2['bc1'],
        pad_row(pools[2], ncp1, np2), e3[0:1], e3[1:2],
        pad_row(pools[3], np2, np3), e4[0:1], e4[1:2],
        params['Wd2'], b2['bd2'], params['Wr12'], b2['br12'],
        params['Wr22'], b2['br22'], params['Wc2'], b2['bc2'],
        params['Wd3'], b2['bd3'], params['Wr13'], b2['br13'],
        params['Wr23'], b2['br23'], params['Wc3'], b2['bc3'])
    outs.extend([v1[:LEVEL_NS[2]], v2[:LEVEL_NS[3]], v3[:LEVEL_NS[4]]])
    return tuple(outs)
